# baseline (speedup 1.0000x reference)
import functools

import jax
import jax.numpy as jnp
from jax import lax
from jax.experimental import pallas as pl
from jax.experimental.pallas import tpu as pltpu
from jax.experimental.pallas import tpu_sc as plsc

N = 10000
E = 160000
D = 256
DE = 16
SEGW = 256

NC, NS, L = 2, 16, 16
HALF = 5120
NPAD = 2 * HALF
SKIP = -1
USE_SKIP = False
DUMMY = NPAD
NOUT = NPAD + 8
WB = HALF // NS

EPW = E // NS
CH = 128
NCHUNK = -(-EPW // CH)
EPW_PAD = NCHUNK * CH
E_PAD = NS * EPW_PAD

_MESH = functools.partial(
    plsc.VectorSubcoreMesh,
    core_axis_name="c", subcore_axis_name="s", num_cores=NC, num_subcores=NS)


def _idx(ref):
    if USE_SKIP:
        return plsc.Indices(ref, ignored_value=SKIP)
    return ref


_BAD = SKIP if USE_SKIP else DUMMY


def _mask_dst(dstb, idxb, c, srcb=None, sidxb=None):
    lo = c * HALF
    for k in range(CH // L):
        sl = pl.ds(k * L, L)
        d = dstb[sl]
        ok = (d >= lo) & (d < lo + HALF)
        idxb[sl] = jnp.where(ok, d, _BAD)
        if srcb is not None:
            sidxb[sl] = jnp.where(ok, srcb[sl], SKIP if USE_SKIP else 0)


def _sc_rows_body(x_hbm, src_hbm, dst_hbm, zd_hbm, s_out,
                  rows, srcb, dstb, idxb, sidxb):
    c = lax.axis_index("c")
    s = lax.axis_index("s")

    pltpu.sync_copy(zd_hbm, rows)
    ob = c * HALF + s * WB
    for t, (o, n) in enumerate(((0, CH), (CH, CH), (2 * CH, WB - 2 * CH))):
        pltpu.sync_copy(rows.at[pl.ds(0, n)], s_out.at[pl.ds(ob + o, n)])
    plsc.subcore_barrier()

    @pl.when(s == 0)
    def _():
        @pl.loop(0, NS * NCHUNK)
        def _(j):
            eb = j * CH
            pltpu.sync_copy(src_hbm.at[pl.ds(eb, CH)], srcb)
            pltpu.sync_copy(dst_hbm.at[pl.ds(eb, CH)], dstb)
            _mask_dst(dstb, idxb, c, srcb, sidxb)
            pltpu.sync_copy(x_hbm.at[_idx(sidxb)], rows)
            pltpu.sync_copy(rows, s_out.at[_idx(idxb)], add=True)


def _sc_seg_body(dst_hbm, el_hbm, zs_hbm, seg_out, elb, dstb, idxb):
    c = lax.axis_index("c")
    s = lax.axis_index("s")
    pltpu.sync_copy(zs_hbm, elb)
    ob = c * HALF + s * WB
    for t, (o, n) in enumerate(((0, CH), (CH, CH), (2 * CH, WB - 2 * CH))):
        pltpu.sync_copy(elb.at[pl.ds(0, n)], seg_out.at[pl.ds(ob + o, n)])
    plsc.subcore_barrier()

    base = s * EPW_PAD

    @pl.loop(0, NCHUNK)
    def _(j):
        eb = base + j * CH
        pltpu.sync_copy(dst_hbm.at[pl.ds(eb, CH)], dstb)
        pltpu.sync_copy(el_hbm.at[pl.ds(eb, CH)], elb)
        _mask_dst(dstb, idxb, c)
        pltpu.sync_copy(elb, seg_out.at[_idx(idxb)], add=True)


def _sc_scatter(x, srcp, dstp, zd):
    out_type = jax.ShapeDtypeStruct((NOUT, D), jnp.float32)
    scratch = [
        pltpu.VMEM((CH, D), jnp.float32),
        pltpu.VMEM((CH,), jnp.int32),
        pltpu.VMEM((CH,), jnp.int32),
        pltpu.VMEM((CH,), jnp.int32),
        pltpu.VMEM((CH,), jnp.int32),
    ]
    fn = pl.kernel(_sc_rows_body, out_type=out_type,
                   mesh=_MESH(), scratch_types=scratch)
    return fn(x, srcp, dstp, zd)


def _sc_seg(dstp, elp, zs):
    out_type = jax.ShapeDtypeStruct((NOUT, SEGW), jnp.float32)
    scratch = [
        pltpu.VMEM((CH, SEGW), jnp.float32),
        pltpu.VMEM((CH,), jnp.int32),
        pltpu.VMEM((CH,), jnp.int32),
    ]
    fn = pl.kernel(_sc_seg_body, out_type=out_type,
                   mesh=_MESH(), scratch_types=scratch)
    return fn(dstp, elp, zs)


BR = 400


def _dense_body(relu, s_ref, seg_ref, x_ref, wl_ref, we_ref, wr_ref, b_ref,
                o_ref):
    seg = seg_ref[...]
    cnt = seg[:, DE:DE + 1]
    dn = (((1,), (1,)), ((), ()))
    agg = (s_ref[...]
           + lax.dot_general(seg[:, :DE], we_ref[...], dn,
                             preferred_element_type=jnp.float32)
           + cnt * b_ref[0:1, :])
    mean = agg / jnp.maximum(cnt, 1.0)
    out = (lax.dot_general(mean, wl_ref[...], dn,
                           preferred_element_type=jnp.float32)
           + b_ref[1:2, :]
           + lax.dot_general(x_ref[...], wr_ref[...], dn,
                             preferred_element_type=jnp.float32))
    o_ref[...] = jnp.maximum(out, 0.0) if relu else out


def _dense(S, seg, x, Wl, We, Wr, bias, relu):
    rowspec = lambda w: pl.BlockSpec((BR, w), lambda i: (i, 0))
    full = lambda a, b: pl.BlockSpec((a, b), lambda i: (0, 0))
    return pl.pallas_call(
        functools.partial(_dense_body, relu),
        grid=(N // BR,),
        in_specs=[rowspec(D), rowspec(SEGW), rowspec(D),
                  full(D, D), full(D, DE), full(D, D), full(8, D)],
        out_specs=rowspec(D),
        out_shape=jax.ShapeDtypeStruct((N, D), jnp.float32),
    )(S, seg, x, Wl, We, Wr, bias)


def kernel(x, edge_index, edge_label, Wl1, bl1, Wr1, We1, be1,
           Wl2, bl2, Wr2, We2, be2):
    src = edge_index[0].astype(jnp.int32)
    dst = edge_index[1].astype(jnp.int32)
    pad = EPW_PAD - EPW
    srcp = jnp.pad(src.reshape(NS, EPW), ((0, 0), (0, pad))).reshape(-1)
    dstp = jnp.pad(dst.reshape(NS, EPW), ((0, 0), (0, pad)),
                   constant_values=NPAD).reshape(-1)
    el_aug = jnp.concatenate(
        [edge_label.astype(jnp.float32),
         jnp.ones((E, 1), jnp.float32),
         jnp.zeros((E, SEGW - DE - 1), jnp.float32)], axis=1)
    elp = jnp.pad(el_aug.reshape(NS, EPW, SEGW),
                  ((0, 0), (0, pad), (0, 0))).reshape(E_PAD, SEGW)
    zd = jnp.zeros((CH, D), jnp.float32)
    zs = jnp.zeros((CH, SEGW), jnp.float32)

    seg = jnp.concatenate(
        [jax.ops.segment_sum(edge_label.astype(jnp.float32), dst,
                             num_segments=N),
         jax.ops.segment_sum(jnp.ones((E, 1), jnp.float32), dst,
                             num_segments=N),
         jnp.zeros((N, SEGW - DE - 1), jnp.float32)], axis=1)
    S1p = _sc_scatter(x, srcp, dstp, zd)
    S1 = S1p[:N]
    bias1 = jnp.zeros((8, D), jnp.float32).at[0].set(be1).at[1].set(bl1)
    bias2 = jnp.zeros((8, D), jnp.float32).at[0].set(be2).at[1].set(bl2)
    h = _dense(S1, seg, x, Wl1, We1, Wr1, bias1, relu=True)
    S2p = _sc_scatter(h, srcp, dstp, zd)
    out = _dense(S2p[:N], seg, h, Wl2, We2, Wr2, bias2, relu=False)
    return out


# speedup vs baseline: 14.9452x; 14.9452x over previous
import dataclasses
import functools

import jax
import jax.numpy as jnp
from jax import lax
from jax.experimental import pallas as pl
from jax.experimental.pallas import tpu as pltpu
from jax.experimental.pallas import tpu_sc as plsc

N = 10000
E = 160000
D = 256
DE = 16
SEGW = 32
ELW = 128

NC, NS, L = 2, 16, 16
NW = NC * NS
RNG = 320
NPAD = NW * RNG
TRASH = RNG
ACCR = RNG + 8

CH = 128
EPAD = 256

_MESH = functools.partial(
    plsc.VectorSubcoreMesh,
    core_axis_name="c", subcore_axis_name="s", num_cores=NC, num_subcores=NS)


def _sc_params():
    cp = pltpu.CompilerParams()
    if "needs_layout_passes" in pltpu.CompilerParams.__dataclass_fields__:
        cp = dataclasses.replace(cp, needs_layout_passes=False)
    return cp

_IOTA = lambda: lax.broadcasted_iota(jnp.int32, (L,), 0)


def _worker_bounds(b0v, b1v):
    c = lax.axis_index("c")
    s = lax.axis_index("s")
    w = c * NS + s
    sel = _IOTA() == s
    w0a, w0b = b0v[pl.ds(0, L)], b0v[pl.ds(L, L)]
    w1a, w1b = b1v[pl.ds(0, L)], b1v[pl.ds(L, L)]
    cz = c == 0
    e0 = jnp.sum(jnp.where(sel & cz, w0a, 0)) + jnp.sum(
        jnp.where(sel & (~cz), w0b, 0))
    e1 = jnp.sum(jnp.where(sel & cz, w1a, 0)) + jnp.sum(
        jnp.where(sel & (~cz), w1b, 0))
    return e0, e1, w * RNG


def _zero_acc(acc, width):
    zero = jnp.zeros((L,), jnp.float32)
    cols = [jnp.full((L,), k * L, jnp.int32) + _IOTA() for k in range(width // L)]

    @pl.loop(0, ACCR)
    def _(r):
        rs = jnp.full((L,), r, jnp.int32)
        for col in cols:
            plsc.store_scatter(acc, [rs, col], zero)


def _mask_chunk(dstb, idxb, rel0, rel1, lo):
    for g in range(CH // L):
        ids = jnp.full((L,), rel0 + g * L, jnp.int32) + _IOTA()
        ok = (ids >= 0) & (ids < rel1)
        idxb[pl.ds(g * L, L)] = jnp.where(
            ok, dstb[pl.ds(g * L, L)] - lo, TRASH)


def _accum_chunk(rows, idxb, acc, width):
    cols = [jnp.full((L,), k * L, jnp.int32) + _IOTA() for k in range(width // L)]

    @pl.loop(0, CH)
    def _(e):
        esp = jnp.full((L,), e, jnp.int32)
        rs = plsc.load_gather(idxb, [esp])
        for col in cols:
            v = plsc.load_gather(rows, [esp, col])
            plsc.addupdate_scatter(acc, [rs, col], v)


def _sc_rows_body(x_hbm, src_hbm, dst_hbm, b0_hbm, b1_hbm, s_out,
                  rows, srcb, dstb, idxb, b0v, b1v, acc):
    pltpu.sync_copy(b0_hbm, b0v)
    pltpu.sync_copy(b1_hbm, b1v)
    e0, e1, lo = _worker_bounds(b0v, b1v)
    _zero_acc(acc, D)

    a0 = (e0 // 8) * 8
    n_ch = (e1 - a0 + CH - 1) // CH

    @pl.loop(0, n_ch)
    def _(j):
        eb = a0 + j * CH
        pltpu.sync_copy(src_hbm.at[pl.ds(eb, CH)], srcb)
        pltpu.sync_copy(dst_hbm.at[pl.ds(eb, CH)], dstb)
        pltpu.sync_copy(x_hbm.at[srcb], rows)
        _mask_chunk(dstb, idxb, eb - e0, e1 - e0, lo)
        _accum_chunk(rows, idxb, acc, D)

    pltpu.sync_copy(acc.at[pl.ds(0, RNG)],
                    s_out.at[pl.ds(lo, RNG)])


def _sc_seg_body(el_hbm, ord_hbm, dst_hbm, b0_hbm, b1_hbm, seg_out,
                 elg, ordb, dstb, idxb, b0v, b1v, acc):
    pltpu.sync_copy(b0_hbm, b0v)
    pltpu.sync_copy(b1_hbm, b1v)
    e0, e1, lo = _worker_bounds(b0v, b1v)
    _zero_acc(acc, SEGW)

    a0 = (e0 // 8) * 8
    n_ch = (e1 - a0 + CH - 1) // CH

    @pl.loop(0, n_ch)
    def _(j):
        eb = a0 + j * CH
        pltpu.sync_copy(ord_hbm.at[pl.ds(eb, CH)], ordb)
        pltpu.sync_copy(dst_hbm.at[pl.ds(eb, CH)], dstb)
        pltpu.sync_copy(el_hbm.at[ordb], elg)
        _mask_chunk(dstb, idxb, eb - e0, e1 - e0, lo)
        _accum_chunk(elg, idxb, acc, SEGW)

    pltpu.sync_copy(acc.at[pl.ds(0, RNG)],
                    seg_out.at[pl.ds(lo, RNG)])


def _sc_scatter(x, ssrc, sdst, b0, b1):
    out_type = jax.ShapeDtypeStruct((NPAD, D), jnp.float32)
    scratch = [
        pltpu.VMEM((CH, D), jnp.float32),
        pltpu.VMEM((CH,), jnp.int32),
        pltpu.VMEM((CH,), jnp.int32),
        pltpu.VMEM((CH,), jnp.int32),
        pltpu.VMEM((2 * L,), jnp.int32),
        pltpu.VMEM((2 * L,), jnp.int32),
        pltpu.VMEM((ACCR, D), jnp.float32),
    ]
    fn = pl.kernel(_sc_rows_body, out_type=out_type,
                   mesh=_MESH(), scratch_types=scratch,
                   compiler_params=_sc_params())
    return fn(x, ssrc, sdst, b0, b1)


def _sc_seg(elp, ordp, sdst, b0, b1):
    out_type = jax.ShapeDtypeStruct((NPAD, SEGW), jnp.float32)
    scratch = [
        pltpu.VMEM((CH, ELW), jnp.float32),
        pltpu.VMEM((CH,), jnp.int32),
        pltpu.VMEM((CH,), jnp.int32),
        pltpu.VMEM((CH,), jnp.int32),
        pltpu.VMEM((2 * L,), jnp.int32),
        pltpu.VMEM((2 * L,), jnp.int32),
        pltpu.VMEM((ACCR, SEGW), jnp.float32),
    ]
    fn = pl.kernel(_sc_seg_body, out_type=out_type,
                   mesh=_MESH(), scratch_types=scratch,
                   compiler_params=_sc_params())
    return fn(elp, ordp, sdst, b0, b1)


BR = 400


def _dense_body(relu, s_ref, seg_ref, x_ref, wl_ref, we_ref, wr_ref, b_ref,
                o_ref):
    seg = seg_ref[...]
    cnt = seg[:, DE:DE + 1]
    dn = (((1,), (1,)), ((), ()))
    agg = (s_ref[...]
           + lax.dot_general(seg[:, :DE], we_ref[...], dn,
                             preferred_element_type=jnp.float32)
           + cnt * b_ref[0:1, :])
    mean = agg / jnp.maximum(cnt, 1.0)
    out = (lax.dot_general(mean, wl_ref[...], dn,
                           preferred_element_type=jnp.float32)
           + b_ref[1:2, :]
           + lax.dot_general(x_ref[...], wr_ref[...], dn,
                             preferred_element_type=jnp.float32))
    o_ref[...] = jnp.maximum(out, 0.0) if relu else out


def _dense(S, seg, x, Wl, We, Wr, bias, relu):
    rowspec = lambda w: pl.BlockSpec((BR, w), lambda i: (i, 0))
    full = lambda a, b: pl.BlockSpec((a, b), lambda i: (0, 0))
    return pl.pallas_call(
        functools.partial(_dense_body, relu),
        grid=(N // BR,),
        in_specs=[rowspec(D), rowspec(SEGW), rowspec(D),
                  full(D, D), full(D, DE), full(D, D), full(8, D)],
        out_specs=rowspec(D),
        out_shape=jax.ShapeDtypeStruct((N, D), jnp.float32),
    )(S, seg, x, Wl, We, Wr, bias)


def kernel(x, edge_index, edge_label, Wl1, bl1, Wr1, We1, be1,
           Wl2, bl2, Wr2, We2, be2):
    src = edge_index[0].astype(jnp.int32)
    dst = edge_index[1].astype(jnp.int32)
    order = jnp.argsort(dst).astype(jnp.int32)
    ssrc = jnp.pad(src[order], (0, EPAD))
    sdst = jnp.pad(dst[order], (0, EPAD))
    ordp = jnp.pad(order, (0, EPAD))
    bounds = jnp.searchsorted(
        sdst[:E], jnp.arange(33, dtype=jnp.int32) * RNG).astype(jnp.int32)
    b0 = jnp.pad(bounds[:32], (0, 0))
    b1 = bounds[1:33]
    elp = jnp.concatenate(
        [edge_label.astype(jnp.float32),
         jnp.ones((E, 1), jnp.float32),
         jnp.zeros((E, ELW - DE - 1), jnp.float32)], axis=1)
    elp = jnp.pad(elp, ((0, EPAD), (0, 0)))

    segp = _sc_seg(elp, ordp, sdst, b0, b1)
    S1p = _sc_scatter(x, ssrc, sdst, b0, b1)
    S1, seg = S1p[:N], segp[:N]
    bias1 = jnp.zeros((8, D), jnp.float32).at[0].set(be1).at[1].set(bl1)
    bias2 = jnp.zeros((8, D), jnp.float32).at[0].set(be2).at[1].set(bl2)
    h = _dense(S1, seg, x, Wl1, We1, Wr1, bias1, relu=True)
    S2p = _sc_scatter(h, ssrc, sdst, b0, b1)
    out = _dense(S2p[:N], seg, h, Wl2, We2, Wr2, bias2, relu=False)
    return out


# speedup vs baseline: 18.0630x; 1.2086x over previous
import dataclasses
import functools

import jax
import jax.numpy as jnp
from jax import lax
from jax.experimental import pallas as pl
from jax.experimental.pallas import tpu as pltpu
from jax.experimental.pallas import tpu_sc as plsc

N = 10000
E = 160000
D = 256
DE = 16
SEGW = 32
ELW = 128

NC, NS, L = 2, 16, 16
NW = NC * NS
RNG = 320
NPAD = NW * RNG
TRASH = RNG
ACCR = RNG + 8

CH = 64
SC_E = 1024
NIC = SC_E // CH
EPAD = 1280

_MESH = functools.partial(
    plsc.VectorSubcoreMesh,
    core_axis_name="c", subcore_axis_name="s", num_cores=NC, num_subcores=NS)


def _sc_params():
    cp = pltpu.CompilerParams()
    if "needs_layout_passes" in pltpu.CompilerParams.__dataclass_fields__:
        cp = dataclasses.replace(cp, needs_layout_passes=False)
    return cp

_IOTA = lambda: lax.broadcasted_iota(jnp.int32, (L,), 0)


def _worker_bounds(b0v, b1v):
    c = lax.axis_index("c")
    s = lax.axis_index("s")
    w = c * NS + s
    sel = _IOTA() == s
    w0a, w0b = b0v[pl.ds(0, L)], b0v[pl.ds(L, L)]
    w1a, w1b = b1v[pl.ds(0, L)], b1v[pl.ds(L, L)]
    cz = c == 0
    e0 = jnp.sum(jnp.where(sel & cz, w0a, 0)) + jnp.sum(
        jnp.where(sel & (~cz), w0b, 0))
    e1 = jnp.sum(jnp.where(sel & cz, w1a, 0)) + jnp.sum(
        jnp.where(sel & (~cz), w1b, 0))
    return e0, e1, w * RNG


def _zero_acc(acc, width):
    zero = jnp.zeros((L,), jnp.float32)
    cols = [jnp.full((L,), k * L, jnp.int32) + _IOTA() for k in range(width // L)]

    @pl.loop(0, ACCR)
    def _(r):
        rs = jnp.full((L,), r, jnp.int32)
        for col in cols:
            plsc.store_scatter(acc, [rs, col], zero)


def _mask_super(dstb, idxb, rel0, rel1, lo):
    for g in range(SC_E // L):
        ids = jnp.full((L,), rel0 + g * L, jnp.int32) + _IOTA()
        ok = (ids >= 0) & (ids < rel1)
        idxb[pl.ds(g * L, L)] = jnp.where(
            ok, dstb[pl.ds(g * L, L)] - lo, TRASH)


def _accum_chunk(rows, idxb, ioff, acc, width):
    cols = [jnp.full((L,), k * L, jnp.int32) + _IOTA() for k in range(width // L)]

    @pl.loop(0, CH)
    def _(e):
        esp = jnp.full((L,), e, jnp.int32)
        rs = plsc.load_gather(idxb, [esp + ioff])
        for col in cols:
            v = plsc.load_gather(rows, [esp, col])
            plsc.addupdate_scatter(acc, [rs, col], v)


def _pipelined_edges(table_hbm, idx_of_edge, dst_hbm, e0, e1, lo,
                     idxsrc, dstb, idxb, bufs, acc, width):
    a0 = (e0 // 8) * 8
    n_sc = (e1 - a0 + SC_E - 1) // SC_E

    @pl.loop(0, n_sc)
    def _(t):
        sb = a0 + t * SC_E
        pltpu.sync_copy(idx_of_edge.at[pl.ds(sb, SC_E)], idxsrc)
        pltpu.sync_copy(dst_hbm.at[pl.ds(sb, SC_E)], dstb)
        _mask_super(dstb, idxb, sb - e0, e1 - e0, lo)

        def gsrc(k):
            return table_hbm.at[idxsrc.at[pl.ds(k * CH, CH)]]

        def issue(k):
            rows_p, sem_p = bufs[k % 2]
            @pl.when(sb + k * CH < e1)
            def _():
                pltpu.async_copy(gsrc(k), rows_p, sem_p)

        def drain(k):
            rows_p, sem_p = bufs[k % 2]
            @pl.when(sb + k * CH < e1)
            def _():
                pltpu.make_async_copy(gsrc(k), rows_p, sem_p).wait()
                _accum_chunk(rows_p, idxb, k * CH, acc, width)

        issue(0)
        for k in range(NIC):
            if k + 1 < NIC:
                issue(k + 1)
            drain(k)


def _sc_rows_body(x_hbm, src_hbm, dst_hbm, b0_hbm, b1_hbm, s_out,
                  rows0, rows1, idxsrc, dstb, idxb, b0v, b1v, acc,
                  sem0, sem1):
    pltpu.sync_copy(b0_hbm, b0v)
    pltpu.sync_copy(b1_hbm, b1v)
    e0, e1, lo = _worker_bounds(b0v, b1v)
    _zero_acc(acc, D)
    _pipelined_edges(x_hbm, src_hbm, dst_hbm, e0, e1, lo,
                     idxsrc, dstb, idxb,
                     ((rows0, sem0), (rows1, sem1)), acc, D)
    pltpu.sync_copy(acc.at[pl.ds(0, RNG)],
                    s_out.at[pl.ds(lo, RNG)])


def _sc_seg_body(el_hbm, ord_hbm, dst_hbm, b0_hbm, b1_hbm, seg_out,
                 elg0, elg1, idxsrc, dstb, idxb, b0v, b1v, acc,
                 sem0, sem1):
    pltpu.sync_copy(b0_hbm, b0v)
    pltpu.sync_copy(b1_hbm, b1v)
    e0, e1, lo = _worker_bounds(b0v, b1v)
    _zero_acc(acc, SEGW)
    _pipelined_edges(el_hbm, ord_hbm, dst_hbm, e0, e1, lo,
                     idxsrc, dstb, idxb,
                     ((elg0, sem0), (elg1, sem1)), acc, SEGW)
    pltpu.sync_copy(acc.at[pl.ds(0, RNG)],
                    seg_out.at[pl.ds(lo, RNG)])


def _sc_scatter(x, ssrc, sdst, b0, b1):
    out_type = jax.ShapeDtypeStruct((NPAD, D), jnp.float32)
    scratch = [
        pltpu.VMEM((CH, D), jnp.float32),
        pltpu.VMEM((CH, D), jnp.float32),
        pltpu.VMEM((SC_E,), jnp.int32),
        pltpu.VMEM((SC_E,), jnp.int32),
        pltpu.VMEM((SC_E,), jnp.int32),
        pltpu.VMEM((2 * L,), jnp.int32),
        pltpu.VMEM((2 * L,), jnp.int32),
        pltpu.VMEM((ACCR, D), jnp.float32),
        pltpu.SemaphoreType.DMA,
        pltpu.SemaphoreType.DMA,
    ]
    fn = pl.kernel(_sc_rows_body, out_type=out_type,
                   mesh=_MESH(), scratch_types=scratch,
                   compiler_params=_sc_params())
    return fn(x, ssrc, sdst, b0, b1)


def _sc_seg(elp, ordp, sdst, b0, b1):
    out_type = jax.ShapeDtypeStruct((NPAD, SEGW), jnp.float32)
    scratch = [
        pltpu.VMEM((CH, ELW), jnp.float32),
        pltpu.VMEM((CH, ELW), jnp.float32),
        pltpu.VMEM((SC_E,), jnp.int32),
        pltpu.VMEM((SC_E,), jnp.int32),
        pltpu.VMEM((SC_E,), jnp.int32),
        pltpu.VMEM((2 * L,), jnp.int32),
        pltpu.VMEM((2 * L,), jnp.int32),
        pltpu.VMEM((ACCR, SEGW), jnp.float32),
        pltpu.SemaphoreType.DMA,
        pltpu.SemaphoreType.DMA,
    ]
    fn = pl.kernel(_sc_seg_body, out_type=out_type,
                   mesh=_MESH(), scratch_types=scratch,
                   compiler_params=_sc_params())
    return fn(elp, ordp, sdst, b0, b1)


BR = 400


def _dense_body(relu, s_ref, seg_ref, x_ref, wl_ref, we_ref, wr_ref, b_ref,
                o_ref):
    seg = seg_ref[...]
    cnt = seg[:, DE:DE + 1]
    dn = (((1,), (1,)), ((), ()))
    agg = (s_ref[...]
           + lax.dot_general(seg[:, :DE], we_ref[...], dn,
                             preferred_element_type=jnp.float32)
           + cnt * b_ref[0:1, :])
    mean = agg / jnp.maximum(cnt, 1.0)
    out = (lax.dot_general(mean, wl_ref[...], dn,
                           preferred_element_type=jnp.float32)
           + b_ref[1:2, :]
           + lax.dot_general(x_ref[...], wr_ref[...], dn,
                             preferred_element_type=jnp.float32))
    o_ref[...] = jnp.maximum(out, 0.0) if relu else out


def _dense(S, seg, x, Wl, We, Wr, bias, relu):
    rowspec = lambda w: pl.BlockSpec((BR, w), lambda i: (i, 0))
    full = lambda a, b: pl.BlockSpec((a, b), lambda i: (0, 0))
    return pl.pallas_call(
        functools.partial(_dense_body, relu),
        grid=(N // BR,),
        in_specs=[rowspec(D), rowspec(SEGW), rowspec(D),
                  full(D, D), full(D, DE), full(D, D), full(8, D)],
        out_specs=rowspec(D),
        out_shape=jax.ShapeDtypeStruct((N, D), jnp.float32),
    )(S, seg, x, Wl, We, Wr, bias)


def kernel(x, edge_index, edge_label, Wl1, bl1, Wr1, We1, be1,
           Wl2, bl2, Wr2, We2, be2):
    src = edge_index[0].astype(jnp.int32)
    dst = edge_index[1].astype(jnp.int32)
    order = jnp.argsort(dst).astype(jnp.int32)
    ssrc = jnp.pad(src[order], (0, EPAD))
    sdst = jnp.pad(dst[order], (0, EPAD))
    ordp = jnp.pad(order, (0, EPAD))
    bounds = jnp.searchsorted(
        sdst[:E], jnp.arange(33, dtype=jnp.int32) * RNG).astype(jnp.int32)
    b0 = jnp.pad(bounds[:32], (0, 0))
    b1 = bounds[1:33]
    elp = jnp.concatenate(
        [edge_label.astype(jnp.float32),
         jnp.ones((E, 1), jnp.float32),
         jnp.zeros((E, ELW - DE - 1), jnp.float32)], axis=1)
    elp = jnp.pad(elp, ((0, EPAD), (0, 0)))

    segp = _sc_seg(elp, ordp, sdst, b0, b1)
    S1p = _sc_scatter(x, ssrc, sdst, b0, b1)
    S1, seg = S1p[:N], segp[:N]
    bias1 = jnp.zeros((8, D), jnp.float32).at[0].set(be1).at[1].set(bl1)
    bias2 = jnp.zeros((8, D), jnp.float32).at[0].set(be2).at[1].set(bl2)
    h = _dense(S1, seg, x, Wl1, We1, Wr1, bias1, relu=True)
    S2p = _sc_scatter(h, ssrc, sdst, b0, b1)
    out = _dense(S2p[:N], seg, h, Wl2, We2, Wr2, bias2, relu=False)
    return out


# speedup vs baseline: 18.6796x; 1.0341x over previous
import dataclasses
import functools

import jax
import jax.numpy as jnp
from jax import lax
from jax.experimental import pallas as pl
from jax.experimental.pallas import tpu as pltpu
from jax.experimental.pallas import tpu_sc as plsc

N = 10000
E = 160000
D = 256
DE = 16
SEGW = 32
ELW = 128

NC, NS, L = 2, 16, 16
NW = NC * NS
RNG = 320
NPAD = NW * RNG
TRASH = RNG
ACCR = RNG + 8

CH = 64
SC_E = 1024
NIC = SC_E // CH
EPAD = 1280

_MESH = functools.partial(
    plsc.VectorSubcoreMesh,
    core_axis_name="c", subcore_axis_name="s", num_cores=NC, num_subcores=NS)


def _sc_params():
    cp = pltpu.CompilerParams()
    if "needs_layout_passes" in pltpu.CompilerParams.__dataclass_fields__:
        cp = dataclasses.replace(cp, needs_layout_passes=False)
    return cp

_IOTA = lambda: lax.broadcasted_iota(jnp.int32, (L,), 0)


def _worker_bounds(b0v, b1v):
    c = lax.axis_index("c")
    s = lax.axis_index("s")
    w = c * NS + s
    sel = _IOTA() == s
    w0a, w0b = b0v[pl.ds(0, L)], b0v[pl.ds(L, L)]
    w1a, w1b = b1v[pl.ds(0, L)], b1v[pl.ds(L, L)]
    cz = c == 0
    e0 = jnp.sum(jnp.where(sel & cz, w0a, 0)) + jnp.sum(
        jnp.where(sel & (~cz), w0b, 0))
    e1 = jnp.sum(jnp.where(sel & cz, w1a, 0)) + jnp.sum(
        jnp.where(sel & (~cz), w1b, 0))
    return e0, e1, w * RNG


def _zero_acc(acc, width):
    zero = jnp.zeros((L,), jnp.float32)
    cols = [jnp.full((L,), k * L, jnp.int32) + _IOTA() for k in range(width // L)]

    @pl.loop(0, ACCR)
    def _(r):
        rs = jnp.full((L,), r, jnp.int32)
        for col in cols:
            plsc.store_scatter(acc, [rs, col], zero)


def _mask_super(dstb, idxb, rel0, rel1, lo):
    for g in range(SC_E // L):
        ids = jnp.full((L,), rel0 + g * L, jnp.int32) + _IOTA()
        ok = (ids >= 0) & (ids < rel1)
        idxb[pl.ds(g * L, L)] = jnp.where(
            ok, dstb[pl.ds(g * L, L)] - lo, TRASH)


def _accum_chunk(rows, idxb, ioff, acc, width):
    cols = [jnp.full((L,), k * L, jnp.int32) + _IOTA() for k in range(width // L)]

    @pl.loop(0, CH, unroll=4)
    def _(e):
        esp = jnp.full((L,), e, jnp.int32)
        rs = plsc.load_gather(idxb, [esp + ioff])
        for col in cols:
            v = plsc.load_gather(rows, [esp, col])
            plsc.addupdate_scatter(acc, [rs, col], v)


def _pipelined_edges(table_hbm, idx_of_edge, dst_hbm, e0, e1, lo,
                     idxsrc, dstb, idxb, bufs, acc, width):
    a0 = (e0 // 8) * 8
    n_sc = (e1 - a0 + SC_E - 1) // SC_E

    @pl.loop(0, n_sc)
    def _(t):
        sb = a0 + t * SC_E
        if idx_of_edge is not None:
            pltpu.sync_copy(idx_of_edge.at[pl.ds(sb, SC_E)], idxsrc)
        pltpu.sync_copy(dst_hbm.at[pl.ds(sb, SC_E)], dstb)
        _mask_super(dstb, idxb, sb - e0, e1 - e0, lo)

        def gsrc(k):
            if idx_of_edge is None:
                return table_hbm.at[pl.ds(sb + k * CH, CH)]
            return table_hbm.at[idxsrc.at[pl.ds(k * CH, CH)]]

        def issue(k):
            rows_p, sem_p = bufs[k % 2]
            @pl.when(sb + k * CH < e1)
            def _():
                pltpu.async_copy(gsrc(k), rows_p, sem_p)

        def drain(k):
            rows_p, sem_p = bufs[k % 2]
            @pl.when(sb + k * CH < e1)
            def _():
                pltpu.make_async_copy(gsrc(k), rows_p, sem_p).wait()
                _accum_chunk(rows_p, idxb, k * CH, acc, width)

        issue(0)
        for k in range(NIC):
            if k + 1 < NIC:
                issue(k + 1)
            drain(k)


def _sc_rows_body(x_hbm, src_hbm, dst_hbm, b0_hbm, b1_hbm, s_out,
                  rows0, rows1, idxsrc, dstb, idxb, b0v, b1v, acc,
                  sem0, sem1):
    pltpu.sync_copy(b0_hbm, b0v)
    pltpu.sync_copy(b1_hbm, b1v)
    e0, e1, lo = _worker_bounds(b0v, b1v)
    _zero_acc(acc, D)
    _pipelined_edges(x_hbm, src_hbm, dst_hbm, e0, e1, lo,
                     idxsrc, dstb, idxb,
                     ((rows0, sem0), (rows1, sem1)), acc, D)
    pltpu.sync_copy(acc.at[pl.ds(0, RNG)],
                    s_out.at[pl.ds(lo, RNG)])


def _sc_seg_body(el_hbm, dst_hbm, b0_hbm, b1_hbm, seg_out,
                 elg0, elg1, dstb, idxb, b0v, b1v, acc,
                 sem0, sem1):
    pltpu.sync_copy(b0_hbm, b0v)
    pltpu.sync_copy(b1_hbm, b1v)
    e0, e1, lo = _worker_bounds(b0v, b1v)
    _zero_acc(acc, SEGW)
    _pipelined_edges(el_hbm, None, dst_hbm, e0, e1, lo,
                     None, dstb, idxb,
                     ((elg0, sem0), (elg1, sem1)), acc, SEGW)
    pltpu.sync_copy(acc.at[pl.ds(0, RNG)],
                    seg_out.at[pl.ds(lo, RNG)])


def _sc_scatter(x, ssrc, sdst, b0, b1):
    out_type = jax.ShapeDtypeStruct((NPAD, D), jnp.float32)
    scratch = [
        pltpu.VMEM((CH, D), jnp.float32),
        pltpu.VMEM((CH, D), jnp.float32),
        pltpu.VMEM((SC_E,), jnp.int32),
        pltpu.VMEM((SC_E,), jnp.int32),
        pltpu.VMEM((SC_E,), jnp.int32),
        pltpu.VMEM((2 * L,), jnp.int32),
        pltpu.VMEM((2 * L,), jnp.int32),
        pltpu.VMEM((ACCR, D), jnp.float32),
        pltpu.SemaphoreType.DMA,
        pltpu.SemaphoreType.DMA,
    ]
    fn = pl.kernel(_sc_rows_body, out_type=out_type,
                   mesh=_MESH(), scratch_types=scratch,
                   compiler_params=_sc_params())
    return fn(x, ssrc, sdst, b0, b1)


def _sc_seg(els, sdst, b0, b1):
    out_type = jax.ShapeDtypeStruct((NPAD, SEGW), jnp.float32)
    scratch = [
        pltpu.VMEM((CH, SEGW), jnp.float32),
        pltpu.VMEM((CH, SEGW), jnp.float32),
        pltpu.VMEM((SC_E,), jnp.int32),
        pltpu.VMEM((SC_E,), jnp.int32),
        pltpu.VMEM((2 * L,), jnp.int32),
        pltpu.VMEM((2 * L,), jnp.int32),
        pltpu.VMEM((ACCR, SEGW), jnp.float32),
        pltpu.SemaphoreType.DMA,
        pltpu.SemaphoreType.DMA,
    ]
    fn = pl.kernel(_sc_seg_body, out_type=out_type,
                   mesh=_MESH(), scratch_types=scratch,
                   compiler_params=_sc_params())
    return fn(els, sdst, b0, b1)


BR = 400


def _dense_body(relu, s_ref, seg_ref, x_ref, wl_ref, we_ref, wr_ref, b_ref,
                o_ref):
    seg = seg_ref[...]
    cnt = seg[:, DE:DE + 1]
    dn = (((1,), (1,)), ((), ()))
    agg = (s_ref[...]
           + lax.dot_general(seg[:, :DE], we_ref[...], dn,
                             preferred_element_type=jnp.float32)
           + cnt * b_ref[0:1, :])
    mean = agg / jnp.maximum(cnt, 1.0)
    out = (lax.dot_general(mean, wl_ref[...], dn,
                           preferred_element_type=jnp.float32)
           + b_ref[1:2, :]
           + lax.dot_general(x_ref[...], wr_ref[...], dn,
                             preferred_element_type=jnp.float32))
    o_ref[...] = jnp.maximum(out, 0.0) if relu else out


def _dense(S, seg, x, Wl, We, Wr, bias, relu):
    rowspec = lambda w: pl.BlockSpec((BR, w), lambda i: (i, 0))
    full = lambda a, b: pl.BlockSpec((a, b), lambda i: (0, 0))
    return pl.pallas_call(
        functools.partial(_dense_body, relu),
        grid=(N // BR,),
        in_specs=[rowspec(D), rowspec(SEGW), rowspec(D),
                  full(D, D), full(D, DE), full(D, D), full(8, D)],
        out_specs=rowspec(D),
        out_shape=jax.ShapeDtypeStruct((N, D), jnp.float32),
    )(S, seg, x, Wl, We, Wr, bias)


def kernel(x, edge_index, edge_label, Wl1, bl1, Wr1, We1, be1,
           Wl2, bl2, Wr2, We2, be2):
    src = edge_index[0].astype(jnp.int32)
    dst = edge_index[1].astype(jnp.int32)
    order = jnp.argsort(dst).astype(jnp.int32)
    ssrc = jnp.pad(src[order], (0, EPAD))
    sdst = jnp.pad(dst[order], (0, EPAD))
    bounds = jnp.searchsorted(
        sdst[:E], jnp.arange(33, dtype=jnp.int32) * RNG).astype(jnp.int32)
    b0 = bounds[:32]
    b1 = bounds[1:33]
    els = jnp.concatenate(
        [edge_label.astype(jnp.float32)[order],
         jnp.ones((E, 1), jnp.float32),
         jnp.zeros((E, SEGW - DE - 1), jnp.float32)], axis=1)
    els = jnp.pad(els, ((0, EPAD), (0, 0)))

    segp = _sc_seg(els, sdst, b0, b1)
    S1p = _sc_scatter(x, ssrc, sdst, b0, b1)
    S1, seg = S1p[:N], segp[:N]
    bias1 = jnp.zeros((8, D), jnp.float32).at[0].set(be1).at[1].set(bl1)
    bias2 = jnp.zeros((8, D), jnp.float32).at[0].set(be2).at[1].set(bl2)
    h = _dense(S1, seg, x, Wl1, We1, Wr1, bias1, relu=True)
    S2p = _sc_scatter(h, ssrc, sdst, b0, b1)
    out = _dense(S2p[:N], seg, h, Wl2, We2, Wr2, bias2, relu=False)
    return out


# speedup vs baseline: 25.8890x; 1.3860x over previous
import dataclasses
import functools

import jax
import jax.numpy as jnp
from jax import lax
from jax.experimental import pallas as pl
from jax.experimental.pallas import tpu as pltpu
from jax.experimental.pallas import tpu_sc as plsc

N = 10000
E = 160000
D = 256
DE = 16
SEGW = 32
ELW = 128

NC, NS, L = 2, 16, 16
NW = NC * NS
RNG = 320
NPAD = NW * RNG
TRASH = RNG
ACCR = RNG + 8

CH = 64
SC_E = 1024
NIC = SC_E // CH
EPAD = 1280

_MESH = functools.partial(
    plsc.VectorSubcoreMesh,
    core_axis_name="c", subcore_axis_name="s", num_cores=NC, num_subcores=NS)


def _sc_params():
    cp = pltpu.CompilerParams()
    if "needs_layout_passes" in pltpu.CompilerParams.__dataclass_fields__:
        cp = dataclasses.replace(cp, needs_layout_passes=False)
    return cp

_IOTA = lambda: lax.broadcasted_iota(jnp.int32, (L,), 0)


def _worker_bounds(b0v, b1v):
    c = lax.axis_index("c")
    s = lax.axis_index("s")
    w = c * NS + s
    sel = _IOTA() == s
    w0a, w0b = b0v[pl.ds(0, L)], b0v[pl.ds(L, L)]
    w1a, w1b = b1v[pl.ds(0, L)], b1v[pl.ds(L, L)]
    cz = c == 0
    e0 = jnp.sum(jnp.where(sel & cz, w0a, 0)) + jnp.sum(
        jnp.where(sel & (~cz), w0b, 0))
    e1 = jnp.sum(jnp.where(sel & cz, w1a, 0)) + jnp.sum(
        jnp.where(sel & (~cz), w1b, 0))
    return e0, e1, w * RNG


def _zero_acc(acc, width):
    zero = jnp.zeros((L,), jnp.float32)
    cols = [jnp.full((L,), k * L, jnp.int32) + _IOTA() for k in range(width // L)]

    @pl.loop(0, ACCR)
    def _(r):
        rs = jnp.full((L,), r, jnp.int32)
        for col in cols:
            plsc.store_scatter(acc, [rs, col], zero)


def _mask_super(dstb, idxb, rel0, rel1, lo):
    for g in range(SC_E // L):
        ids = jnp.full((L,), rel0 + g * L, jnp.int32) + _IOTA()
        ok = (ids >= 0) & (ids < rel1)
        idxb[pl.ds(g * L, L)] = jnp.where(
            ok, dstb[pl.ds(g * L, L)] - lo, TRASH)


def _accum_chunk(rows, idxb, ioff, acc, width):
    cols = [jnp.full((L,), k * L, jnp.int32) + _IOTA() for k in range(width // L)]

    @pl.loop(0, CH, unroll=2)
    def _(e):
        esp = jnp.full((L,), e, jnp.int32)
        rs = plsc.load_gather(idxb, [esp + ioff])
        for ci in range(0, len(cols), 4):
            grp = cols[ci:ci + 4]
            vals = [plsc.load_gather(rows, [esp, col]) for col in grp]
            for col, v in zip(grp, vals):
                plsc.addupdate_scatter(acc, [rs, col], v)


def _pipelined_edges(table_hbm, idx_of_edge, dst_hbm, e0, e1, lo,
                     idxsrc, dstb, idxb, bufs, acc, width):
    a0 = (e0 // 8) * 8
    n_sc = (e1 - a0 + SC_E - 1) // SC_E

    @pl.loop(0, n_sc)
    def _(t):
        sb = a0 + t * SC_E
        if idx_of_edge is not None:
            pltpu.sync_copy(idx_of_edge.at[pl.ds(sb, SC_E)], idxsrc)
        pltpu.sync_copy(dst_hbm.at[pl.ds(sb, SC_E)], dstb)
        _mask_super(dstb, idxb, sb - e0, e1 - e0, lo)

        def gsrc(k):
            if idx_of_edge is None:
                return table_hbm.at[pl.ds(sb + k * CH, CH)]
            return table_hbm.at[idxsrc.at[pl.ds(k * CH, CH)]]

        def issue(k):
            rows_p, sem_p = bufs[k % 2]
            @pl.when(sb + k * CH < e1)
            def _():
                pltpu.async_copy(gsrc(k), rows_p, sem_p)

        def drain(k):
            rows_p, sem_p = bufs[k % 2]
            @pl.when(sb + k * CH < e1)
            def _():
                pltpu.make_async_copy(gsrc(k), rows_p, sem_p).wait()
                _accum_chunk(rows_p, idxb, k * CH, acc, width)

        issue(0)
        for k in range(NIC):
            if k + 1 < NIC:
                issue(k + 1)
            drain(k)


def _sc_rows_body(x_hbm, src_hbm, dst_hbm, b0_hbm, b1_hbm, s_out,
                  rows0, rows1, idxsrc, dstb, idxb, b0v, b1v, acc,
                  sem0, sem1):
    pltpu.sync_copy(b0_hbm, b0v)
    pltpu.sync_copy(b1_hbm, b1v)
    e0, e1, lo = _worker_bounds(b0v, b1v)
    _zero_acc(acc, D)
    _pipelined_edges(x_hbm, src_hbm, dst_hbm, e0, e1, lo,
                     idxsrc, dstb, idxb,
                     ((rows0, sem0), (rows1, sem1)), acc, D)
    pltpu.sync_copy(acc.at[pl.ds(0, RNG)],
                    s_out.at[pl.ds(lo, RNG)])


def _sc_seg_body(el_hbm, dst_hbm, b0_hbm, b1_hbm, seg_out,
                 elg0, elg1, dstb, idxb, b0v, b1v, acc,
                 sem0, sem1):
    pltpu.sync_copy(b0_hbm, b0v)
    pltpu.sync_copy(b1_hbm, b1v)
    e0, e1, lo = _worker_bounds(b0v, b1v)
    _zero_acc(acc, SEGW)
    _pipelined_edges(el_hbm, None, dst_hbm, e0, e1, lo,
                     None, dstb, idxb,
                     ((elg0, sem0), (elg1, sem1)), acc, SEGW)
    pltpu.sync_copy(acc.at[pl.ds(0, RNG)],
                    seg_out.at[pl.ds(lo, RNG)])


def _sc_scatter(x, ssrc, sdst, b0, b1):
    out_type = jax.ShapeDtypeStruct((NPAD, D), jnp.float32)
    scratch = [
        pltpu.VMEM((CH, D), jnp.float32),
        pltpu.VMEM((CH, D), jnp.float32),
        pltpu.VMEM((SC_E,), jnp.int32),
        pltpu.VMEM((SC_E,), jnp.int32),
        pltpu.VMEM((SC_E,), jnp.int32),
        pltpu.VMEM((2 * L,), jnp.int32),
        pltpu.VMEM((2 * L,), jnp.int32),
        pltpu.VMEM((ACCR, D), jnp.float32),
        pltpu.SemaphoreType.DMA,
        pltpu.SemaphoreType.DMA,
    ]
    fn = pl.kernel(_sc_rows_body, out_type=out_type,
                   mesh=_MESH(), scratch_types=scratch,
                   compiler_params=_sc_params())
    return fn(x, ssrc, sdst, b0, b1)


def _sc_seg(els, sdst, b0, b1):
    out_type = jax.ShapeDtypeStruct((NPAD, SEGW), jnp.float32)
    scratch = [
        pltpu.VMEM((CH, SEGW), jnp.float32),
        pltpu.VMEM((CH, SEGW), jnp.float32),
        pltpu.VMEM((SC_E,), jnp.int32),
        pltpu.VMEM((SC_E,), jnp.int32),
        pltpu.VMEM((2 * L,), jnp.int32),
        pltpu.VMEM((2 * L,), jnp.int32),
        pltpu.VMEM((ACCR, SEGW), jnp.float32),
        pltpu.SemaphoreType.DMA,
        pltpu.SemaphoreType.DMA,
    ]
    fn = pl.kernel(_sc_seg_body, out_type=out_type,
                   mesh=_MESH(), scratch_types=scratch,
                   compiler_params=_sc_params())
    return fn(els, sdst, b0, b1)


BR = 400


def _dense_body(relu, s_ref, seg_ref, x_ref, wl_ref, we_ref, wr_ref, b_ref,
                o_ref):
    seg = seg_ref[...]
    cnt = seg[:, DE:DE + 1]
    dn = (((1,), (1,)), ((), ()))
    agg = (s_ref[...]
           + lax.dot_general(seg[:, :DE], we_ref[...], dn,
                             preferred_element_type=jnp.float32)
           + cnt * b_ref[0:1, :])
    mean = agg / jnp.maximum(cnt, 1.0)
    out = (lax.dot_general(mean, wl_ref[...], dn,
                           preferred_element_type=jnp.float32)
           + b_ref[1:2, :]
           + lax.dot_general(x_ref[...], wr_ref[...], dn,
                             preferred_element_type=jnp.float32))
    o_ref[...] = jnp.maximum(out, 0.0) if relu else out


def _dense(S, seg, x, Wl, We, Wr, bias, relu):
    rowspec = lambda w: pl.BlockSpec((BR, w), lambda i: (i, 0))
    full = lambda a, b: pl.BlockSpec((a, b), lambda i: (0, 0))
    return pl.pallas_call(
        functools.partial(_dense_body, relu),
        grid=(N // BR,),
        in_specs=[rowspec(D), rowspec(SEGW), rowspec(D),
                  full(D, D), full(D, DE), full(D, D), full(8, D)],
        out_specs=rowspec(D),
        out_shape=jax.ShapeDtypeStruct((N, D), jnp.float32),
    )(S, seg, x, Wl, We, Wr, bias)


def kernel(x, edge_index, edge_label, Wl1, bl1, Wr1, We1, be1,
           Wl2, bl2, Wr2, We2, be2):
    src = edge_index[0].astype(jnp.int32)
    dst = edge_index[1].astype(jnp.int32)
    order = jnp.argsort(dst).astype(jnp.int32)
    ssrc = jnp.pad(src[order], (0, EPAD))
    sdst = jnp.pad(dst[order], (0, EPAD))
    bounds = jnp.searchsorted(
        sdst[:E], jnp.arange(33, dtype=jnp.int32) * RNG).astype(jnp.int32)
    b0 = bounds[:32]
    b1 = bounds[1:33]
    els = jnp.concatenate(
        [edge_label.astype(jnp.float32)[order],
         jnp.ones((E, 1), jnp.float32),
         jnp.zeros((E, SEGW - DE - 1), jnp.float32)], axis=1)
    els = jnp.pad(els, ((0, EPAD), (0, 0)))

    segp = _sc_seg(els, sdst, b0, b1)
    S1p = _sc_scatter(x, ssrc, sdst, b0, b1)
    S1, seg = S1p[:N], segp[:N]
    bias1 = jnp.zeros((8, D), jnp.float32).at[0].set(be1).at[1].set(bl1)
    bias2 = jnp.zeros((8, D), jnp.float32).at[0].set(be2).at[1].set(bl2)
    h = _dense(S1, seg, x, Wl1, We1, Wr1, bias1, relu=True)
    S2p = _sc_scatter(h, ssrc, sdst, b0, b1)
    out = _dense(S2p[:N], seg, h, Wl2, We2, Wr2, bias2, relu=False)
    return out


# speedup vs baseline: 27.3233x; 1.0554x over previous
import dataclasses
import functools

import jax
import jax.numpy as jnp
from jax import lax
from jax.experimental import pallas as pl
from jax.experimental.pallas import tpu as pltpu
from jax.experimental.pallas import tpu_sc as plsc

N = 10000
E = 160000
D = 256
DE = 16
SEGW = 32
ELW = 128

NC, NS, L = 2, 16, 16
NW = NC * NS
RNG = 320
NPAD = NW * RNG
TRASH = RNG
ACCR = RNG + 8

CH = 64
SC_E = 1024
NIC = SC_E // CH
EPAD = 1280

_MESH = functools.partial(
    plsc.VectorSubcoreMesh,
    core_axis_name="c", subcore_axis_name="s", num_cores=NC, num_subcores=NS)


def _sc_params():
    cp = pltpu.CompilerParams()
    if "needs_layout_passes" in pltpu.CompilerParams.__dataclass_fields__:
        cp = dataclasses.replace(cp, needs_layout_passes=False)
    return cp

_IOTA = lambda: lax.broadcasted_iota(jnp.int32, (L,), 0)


def _worker_bounds(b0v, b1v):
    c = lax.axis_index("c")
    s = lax.axis_index("s")
    w = c * NS + s
    sel = _IOTA() == s
    w0a, w0b = b0v[pl.ds(0, L)], b0v[pl.ds(L, L)]
    w1a, w1b = b1v[pl.ds(0, L)], b1v[pl.ds(L, L)]
    cz = c == 0
    e0 = jnp.sum(jnp.where(sel & cz, w0a, 0)) + jnp.sum(
        jnp.where(sel & (~cz), w0b, 0))
    e1 = jnp.sum(jnp.where(sel & cz, w1a, 0)) + jnp.sum(
        jnp.where(sel & (~cz), w1b, 0))
    return e0, e1, w * RNG


def _zero_acc(acc, width):
    zero = jnp.zeros((L,), jnp.float32)
    cols = [jnp.full((L,), k * L, jnp.int32) + _IOTA() for k in range(width // L)]

    @pl.loop(0, ACCR)
    def _(r):
        rs = jnp.full((L,), r, jnp.int32)
        for col in cols:
            plsc.store_scatter(acc, [rs, col], zero)


def _mask_super(dstb, idxb, rel0, rel1, lo):
    for g in range(SC_E // L):
        ids = jnp.full((L,), rel0 + g * L, jnp.int32) + _IOTA()
        ok = (ids >= 0) & (ids < rel1)
        idxb[pl.ds(g * L, L)] = jnp.where(
            ok, dstb[pl.ds(g * L, L)] - lo, TRASH)


def _accum_chunk(rows, idxb, ioff, acc, width):
    cols = [jnp.full((L,), k * L, jnp.int32) + _IOTA() for k in range(width // L)]

    @pl.loop(0, CH, unroll=2)
    def _(e):
        esp = jnp.full((L,), e, jnp.int32)
        rs = plsc.load_gather(idxb, [esp + ioff])
        for ci in range(0, len(cols), 8):
            grp = cols[ci:ci + 8]
            vals = [plsc.load_gather(rows, [esp, col]) for col in grp]
            for col, v in zip(grp, vals):
                plsc.addupdate_scatter(acc, [rs, col], v)


def _pipelined_edges(table_hbm, idx_of_edge, dst_hbm, e0, e1, lo,
                     idxsrc, dstb, idxb, bufs, acc, width):
    a0 = (e0 // 8) * 8
    n_sc = (e1 - a0 + SC_E - 1) // SC_E

    @pl.loop(0, n_sc)
    def _(t):
        sb = a0 + t * SC_E
        if idx_of_edge is not None:
            pltpu.sync_copy(idx_of_edge.at[pl.ds(sb, SC_E)], idxsrc)
        pltpu.sync_copy(dst_hbm.at[pl.ds(sb, SC_E)], dstb)
        _mask_super(dstb, idxb, sb - e0, e1 - e0, lo)

        def gsrc(k):
            if idx_of_edge is None:
                return table_hbm.at[pl.ds(sb + k * CH, CH)]
            return table_hbm.at[idxsrc.at[pl.ds(k * CH, CH)]]

        def issue(k):
            rows_p, sem_p = bufs[k % 2]
            @pl.when(sb + k * CH < e1)
            def _():
                pltpu.async_copy(gsrc(k), rows_p, sem_p)

        def drain(k):
            rows_p, sem_p = bufs[k % 2]
            @pl.when(sb + k * CH < e1)
            def _():
                pltpu.make_async_copy(gsrc(k), rows_p, sem_p).wait()
                _accum_chunk(rows_p, idxb, k * CH, acc, width)

        issue(0)
        for k in range(NIC):
            if k + 1 < NIC:
                issue(k + 1)
            drain(k)


def _sc_rows_body(x_hbm, src_hbm, dst_hbm, b0_hbm, b1_hbm, s_out,
                  rows0, rows1, idxsrc, dstb, idxb, b0v, b1v, acc,
                  sem0, sem1):
    pltpu.sync_copy(b0_hbm, b0v)
    pltpu.sync_copy(b1_hbm, b1v)
    e0, e1, lo = _worker_bounds(b0v, b1v)
    _zero_acc(acc, D)
    _pipelined_edges(x_hbm, src_hbm, dst_hbm, e0, e1, lo,
                     idxsrc, dstb, idxb,
                     ((rows0, sem0), (rows1, sem1)), acc, D)
    pltpu.sync_copy(acc.at[pl.ds(0, RNG)],
                    s_out.at[pl.ds(lo, RNG)])


def _sc_seg_body(el_hbm, dst_hbm, b0_hbm, b1_hbm, seg_out,
                 elg0, elg1, dstb, idxb, b0v, b1v, acc,
                 sem0, sem1):
    pltpu.sync_copy(b0_hbm, b0v)
    pltpu.sync_copy(b1_hbm, b1v)
    e0, e1, lo = _worker_bounds(b0v, b1v)
    _zero_acc(acc, SEGW)
    _pipelined_edges(el_hbm, None, dst_hbm, e0, e1, lo,
                     None, dstb, idxb,
                     ((elg0, sem0), (elg1, sem1)), acc, SEGW)
    pltpu.sync_copy(acc.at[pl.ds(0, RNG)],
                    seg_out.at[pl.ds(lo, RNG)])


def _sc_scatter(x, ssrc, sdst, b0, b1):
    out_type = jax.ShapeDtypeStruct((NPAD, D), jnp.float32)
    scratch = [
        pltpu.VMEM((CH, D), jnp.float32),
        pltpu.VMEM((CH, D), jnp.float32),
        pltpu.VMEM((SC_E,), jnp.int32),
        pltpu.VMEM((SC_E,), jnp.int32),
        pltpu.VMEM((SC_E,), jnp.int32),
        pltpu.VMEM((2 * L,), jnp.int32),
        pltpu.VMEM((2 * L,), jnp.int32),
        pltpu.VMEM((ACCR, D), jnp.float32),
        pltpu.SemaphoreType.DMA,
        pltpu.SemaphoreType.DMA,
    ]
    fn = pl.kernel(_sc_rows_body, out_type=out_type,
                   mesh=_MESH(), scratch_types=scratch,
                   compiler_params=_sc_params())
    return fn(x, ssrc, sdst, b0, b1)


def _sc_seg(els, sdst, b0, b1):
    out_type = jax.ShapeDtypeStruct((NPAD, SEGW), jnp.float32)
    scratch = [
        pltpu.VMEM((CH, SEGW), jnp.float32),
        pltpu.VMEM((CH, SEGW), jnp.float32),
        pltpu.VMEM((SC_E,), jnp.int32),
        pltpu.VMEM((SC_E,), jnp.int32),
        pltpu.VMEM((2 * L,), jnp.int32),
        pltpu.VMEM((2 * L,), jnp.int32),
        pltpu.VMEM((ACCR, SEGW), jnp.float32),
        pltpu.SemaphoreType.DMA,
        pltpu.SemaphoreType.DMA,
    ]
    fn = pl.kernel(_sc_seg_body, out_type=out_type,
                   mesh=_MESH(), scratch_types=scratch,
                   compiler_params=_sc_params())
    return fn(els, sdst, b0, b1)


BR = 512


def _dense_body(relu, s_ref, seg_ref, x_ref, wl_ref, we_ref, wr_ref, b_ref,
                o_ref):
    seg = seg_ref[...]
    cnt = seg[:, DE:DE + 1]
    dn = (((1,), (1,)), ((), ()))
    agg = (s_ref[...]
           + lax.dot_general(seg[:, :DE], we_ref[...], dn,
                             preferred_element_type=jnp.float32)
           + cnt * b_ref[0:1, :])
    mean = agg / jnp.maximum(cnt, 1.0)
    out = (lax.dot_general(mean, wl_ref[...], dn,
                           preferred_element_type=jnp.float32)
           + b_ref[1:2, :]
           + lax.dot_general(x_ref[...], wr_ref[...], dn,
                             preferred_element_type=jnp.float32))
    o_ref[...] = jnp.maximum(out, 0.0) if relu else out


def _dense(S, seg, x, Wl, We, Wr, bias, relu):
    rowspec = lambda w: pl.BlockSpec((BR, w), lambda i: (i, 0))
    full = lambda a, b: pl.BlockSpec((a, b), lambda i: (0, 0))
    return pl.pallas_call(
        functools.partial(_dense_body, relu),
        grid=(NPAD // BR,),
        in_specs=[rowspec(D), rowspec(SEGW), rowspec(D),
                  full(D, D), full(D, DE), full(D, D), full(8, D)],
        out_specs=rowspec(D),
        out_shape=jax.ShapeDtypeStruct((NPAD, D), jnp.float32),
    )(S, seg, x, Wl, We, Wr, bias)


def kernel(x, edge_index, edge_label, Wl1, bl1, Wr1, We1, be1,
           Wl2, bl2, Wr2, We2, be2):
    src = edge_index[0].astype(jnp.int32)
    dst = edge_index[1].astype(jnp.int32)
    order = jnp.argsort(dst).astype(jnp.int32)
    ssrc = jnp.pad(src[order], (0, EPAD))
    sdst = jnp.pad(dst[order], (0, EPAD))
    bounds = jnp.searchsorted(
        sdst[:E], jnp.arange(33, dtype=jnp.int32) * RNG).astype(jnp.int32)
    b0 = bounds[:32]
    b1 = bounds[1:33]
    els = jnp.concatenate(
        [edge_label.astype(jnp.float32)[order],
         jnp.ones((E, 1), jnp.float32),
         jnp.zeros((E, SEGW - DE - 1), jnp.float32)], axis=1)
    els = jnp.pad(els, ((0, EPAD), (0, 0)))

    xpad = jnp.pad(x, ((0, NPAD - N), (0, 0)))
    segp = _sc_seg(els, sdst, b0, b1)
    S1p = _sc_scatter(x, ssrc, sdst, b0, b1)
    bias1 = jnp.zeros((8, D), jnp.float32).at[0].set(be1).at[1].set(bl1)
    bias2 = jnp.zeros((8, D), jnp.float32).at[0].set(be2).at[1].set(bl2)
    h = _dense(S1p, segp, xpad, Wl1, We1, Wr1, bias1, relu=True)
    S2p = _sc_scatter(h, ssrc, sdst, b0, b1)
    out = _dense(S2p, segp, h, Wl2, We2, Wr2, bias2, relu=False)
    return out[:N]
